# baseline (speedup 1.0000x reference)
import functools
import math

import jax
import jax.numpy as jnp
from jax import lax
from jax.experimental import pallas as pl
from jax.experimental.pallas import tpu as pltpu
from jax.experimental.pallas import tpu_sc as plsc

_NC, _NS, _L = 2, 16, 16
_NW = _NC * _NS
_LOG_2PI = math.log(2.0 * math.pi)


def _lse_body(x_ref, o_ref):
    x = x_ref[...]
    m = jnp.max(x)
    s = jnp.sum(jnp.exp(x - m))
    o_ref[...] = jnp.full(o_ref.shape, m + jnp.log(s), o_ref.dtype)


def _compute_lse(logits):
    n = logits.shape[0]
    x = logits.reshape(100, n // 100)
    return pl.pallas_call(
        _lse_body,
        out_shape=jax.ShapeDtypeStruct((128,), jnp.float32),
    )(x)


def _make_sc_kernel(B):
    bpw = B // _NW
    nchunk = bpw // 128
    niter = bpw // _L

    @functools.partial(
        pl.kernel,
        out_type=jax.ShapeDtypeStruct((B,), jnp.float32),
        mesh=plsc.VectorSubcoreMesh(core_axis_name="c", subcore_axis_name="s"),
        compiler_params=pltpu.CompilerParams(
            use_tc_tiling_on_sc=False, needs_layout_passes=False
        ),
        scratch_types=[
            pltpu.VMEM((bpw * 3,), jnp.float32),
            pltpu.VMEM((nchunk, 128), jnp.int32),
            pltpu.VMEM((nchunk, 128), jnp.int32),
            pltpu.VMEM((nchunk, 128), jnp.int32),
            pltpu.VMEM((bpw,), jnp.float32),
            pltpu.VMEM((bpw,), jnp.float32),
            pltpu.VMEM((bpw,), jnp.float32),
            pltpu.VMEM((bpw,), jnp.float32),
            pltpu.VMEM((bpw,), jnp.float32),
            pltpu.VMEM((bpw,), jnp.float32),
            pltpu.VMEM((_L,), jnp.float32),
            pltpu.SemaphoreType.DMA,
            pltpu.SemaphoreType.DMA,
        ],
    )
    def sc_kernel(action_hbm, td_hbm, ls_hbm, mu_hbm, lse_hbm, out_hbm,
                  act_v, it_v, i2t_v, i2t1_v,
                  td_v, ls0_v, ls1_v, mu0_v, mu1_v, out_v, lse_v,
                  sem, csem):
        wid = lax.axis_index("s") * _NC + lax.axis_index("c")
        base = wid * bpw
        cpl = pltpu.async_copy(lse_hbm.at[pl.ds(0, _L)], lse_v, csem)
        pltpu.sync_copy(action_hbm.at[pl.ds(base * 3, bpw * 3)], act_v)

        for i in range(niter):
            j, off = divmod(i * _L, 128)
            rows3 = lax.iota(jnp.int32, _L) * 3 + i * (3 * _L)
            t = plsc.load_gather(act_v, [rows3]).astype(jnp.int32)
            it_v[j, pl.ds(off, _L)] = t
            i2t_v[j, pl.ds(off, _L)] = t + t
            i2t1_v[j, pl.ds(off, _L)] = t + t + 1

        copies = []
        for j in range(nchunk):
            sl = pl.ds(j * 128, 128)
            copies.append(pltpu.async_copy(td_hbm.at[it_v.at[j]], td_v.at[sl], sem))
            copies.append(pltpu.async_copy(ls_hbm.at[i2t_v.at[j]], ls0_v.at[sl], sem))
            copies.append(pltpu.async_copy(ls_hbm.at[i2t1_v.at[j]], ls1_v.at[sl], sem))
            copies.append(pltpu.async_copy(mu_hbm.at[i2t_v.at[j]], mu0_v.at[sl], sem))
            copies.append(pltpu.async_copy(mu_hbm.at[i2t1_v.at[j]], mu1_v.at[sl], sem))
        for cp in copies:
            cp.wait()
        cpl.wait()

        lse = lse_v[...]
        for i in range(niter):
            sl = pl.ds(i * _L, _L)
            rows3 = lax.iota(jnp.int32, _L) * 3 + i * (3 * _L)
            p0 = plsc.load_gather(act_v, [rows3 + 1])
            p1 = plsc.load_gather(act_v, [rows3 + 2])
            d0 = p0 - mu0_v[sl]
            d1 = p1 - mu1_v[sl]
            ls0 = ls0_v[sl]
            ls1 = ls1_v[sl]
            q = d0 * d0 * jnp.exp(-ls0) + d1 * d1 * jnp.exp(-ls1)
            out_v[sl] = td_v[sl] - lse - 0.5 * (q + ls0 + ls1) - _LOG_2PI

        pltpu.sync_copy(out_v, out_hbm.at[pl.ds(base, bpw)])

    return sc_kernel


def kernel(action, tool_distribution, log_std, means):
    lse = _compute_lse(tool_distribution)
    sc = _make_sc_kernel(action.shape[0])
    return sc(action.reshape(-1), tool_distribution, log_std.reshape(-1),
              means.reshape(-1), lse)
